# Initial kernel scaffold; baseline (speedup 1.0000x reference)
import jax
import jax.numpy as jnp
from jax.experimental import pallas as pl


def kernel(h, h2, bc, W0, b0, W1):
    raise NotImplementedError("write your pallas kernel here")


# speedup vs baseline: 58.2078x; 58.2078x over previous
import jax
import jax.numpy as jnp
from jax.experimental import pallas as pl
from jax.experimental.pallas import tpu as pltpu

_H = 2048
_D = 512
_NH = 16
_TPAD = 2 * _H
_BLK_COLS = 1024
_BLK_ROWS = 256


def _mlp_kernel(c_ref, w0_ref, b0_ref, w1t_ref, o_ref):
    r = jnp.maximum(w0_ref[...] * c_ref[...] + b0_ref[...], 0.0)
    t = jax.lax.dot_general(
        w1t_ref[...], r, (((1,), (0,)), ((), ())),
        preferred_element_type=jnp.float32,
        precision=jax.lax.Precision.HIGHEST,
    )
    o_ref[...] = 16.0 * jax.nn.sigmoid(t)


def _toeplitz_kernel(t_ref, o_ref):
    i0 = pl.program_id(1) * _BLK_ROWS
    x = jnp.broadcast_to(t_ref[0], (_BLK_ROWS, _TPAD))
    y = pltpu.roll(x, _TPAD // 2 + 1, axis=1, stride=1, stride_axis=0)
    z = pltpu.roll(y, i0, axis=1)
    o_ref[...] = z[None, :, :_H]


def kernel(h, h2, bc, W0, b0, W1):
    f32 = jnp.float32
    coords_open = jnp.arange(-(_H - 1), _H, dtype=f32) / (h - 1)
    periodic_parts = jnp.concatenate([
        jnp.arange(1, _H // 2 + 1, dtype=f32),
        jnp.arange(-(_H // 2 - 1), _H // 2 + 1, dtype=f32),
        jnp.arange(-(_H // 2 - 1), 0, dtype=f32),
    ]) / (h - 1)
    pad_len = 2 * _H - 1 - periodic_parts.shape[0]
    coords_periodic = jnp.concatenate(
        [periodic_parts, jnp.zeros(pad_len, dtype=f32)])
    rel = jnp.where(bc == 1, coords_periodic, coords_open)
    c_pad = jnp.concatenate([rel, jnp.zeros(1, dtype=f32)]).reshape(1, _TPAD)

    w0c = W0.reshape(1, _D).T
    b0c = b0.reshape(_D, 1)
    w1t = W1.reshape(_D, _NH).T

    t_pad = pl.pallas_call(
        _mlp_kernel,
        grid=(_TPAD // _BLK_COLS,),
        in_specs=[
            pl.BlockSpec((1, _BLK_COLS), lambda j: (0, j)),
            pl.BlockSpec((_D, 1), lambda j: (0, 0)),
            pl.BlockSpec((_D, 1), lambda j: (0, 0)),
            pl.BlockSpec((_NH, _D), lambda j: (0, 0)),
        ],
        out_specs=pl.BlockSpec((_NH, _BLK_COLS), lambda j: (0, j)),
        out_shape=jax.ShapeDtypeStruct((_NH, _TPAD), f32),
    )(c_pad, w0c, b0c, w1t)
    t_pad = t_pad.reshape(_NH, 1, _TPAD)

    out = pl.pallas_call(
        _toeplitz_kernel,
        grid=(_NH, _H // _BLK_ROWS),
        in_specs=[pl.BlockSpec((1, 1, _TPAD), lambda n, ib: (n, 0, 0))],
        out_specs=pl.BlockSpec((1, _BLK_ROWS, _H), lambda n, ib: (n, ib, 0)),
        out_shape=jax.ShapeDtypeStruct((_NH, _H, _H), f32),
    )(t_pad)
    return out[None]


# speedup vs baseline: 128.0132x; 2.1992x over previous
import jax
import jax.numpy as jnp
from jax.experimental import pallas as pl
from jax.experimental.pallas import tpu as pltpu

_H = 2048
_D = 512
_NH = 16
_TPAD = 2 * _H
_BLK_COLS = 1024
_BLK_ROWS = 256


def _mlp_kernel(c_ref, w0_ref, b0_ref, w1t_ref, o_ref):
    r = jnp.maximum(w0_ref[...] * c_ref[...] + b0_ref[...], 0.0)
    t = jax.lax.dot_general(
        w1t_ref[...], r, (((1,), (0,)), ((), ())),
        preferred_element_type=jnp.float32,
        precision=jax.lax.Precision.HIGHEST,
    )
    o_ref[...] = 16.0 * jax.nn.sigmoid(t)


_SLAB = _H + _BLK_ROWS


def _toeplitz_kernel(t_ref, o_ref):
    i0 = pl.program_id(1) * _BLK_ROWS
    slab = t_ref[0, :, pl.ds(_H - _BLK_ROWS - i0, _SLAB)]
    x = jnp.broadcast_to(slab, (_BLK_ROWS, _SLAB))
    y = pltpu.roll(x, _SLAB - (_BLK_ROWS - 1), axis=1, stride=1,
                   stride_axis=0)
    o_ref[...] = y[None, :, :_H]


def kernel(h, h2, bc, W0, b0, W1):
    f32 = jnp.float32
    coords_open = jnp.arange(-(_H - 1), _H, dtype=f32) / (h - 1)
    periodic_parts = jnp.concatenate([
        jnp.arange(1, _H // 2 + 1, dtype=f32),
        jnp.arange(-(_H // 2 - 1), _H // 2 + 1, dtype=f32),
        jnp.arange(-(_H // 2 - 1), 0, dtype=f32),
    ]) / (h - 1)
    pad_len = 2 * _H - 1 - periodic_parts.shape[0]
    coords_periodic = jnp.concatenate(
        [periodic_parts, jnp.zeros(pad_len, dtype=f32)])
    rel = jnp.where(bc == 1, coords_periodic, coords_open)
    c_pad = jnp.concatenate([rel, jnp.zeros(1, dtype=f32)]).reshape(1, _TPAD)

    w0c = W0.reshape(1, _D).T
    b0c = b0.reshape(_D, 1)
    w1t = W1.reshape(_D, _NH).T

    t_pad = pl.pallas_call(
        _mlp_kernel,
        grid=(_TPAD // _BLK_COLS,),
        in_specs=[
            pl.BlockSpec((1, _BLK_COLS), lambda j: (0, j)),
            pl.BlockSpec((_D, 1), lambda j: (0, 0)),
            pl.BlockSpec((_D, 1), lambda j: (0, 0)),
            pl.BlockSpec((_NH, _D), lambda j: (0, 0)),
        ],
        out_specs=pl.BlockSpec((_NH, _BLK_COLS), lambda j: (0, j)),
        out_shape=jax.ShapeDtypeStruct((_NH, _TPAD), f32),
    )(c_pad, w0c, b0c, w1t)
    t_pad = t_pad.reshape(_NH, 1, _TPAD)

    out = pl.pallas_call(
        _toeplitz_kernel,
        grid=(_NH, _H // _BLK_ROWS),
        in_specs=[pl.BlockSpec((1, 1, _TPAD), lambda n, ib: (n, 0, 0))],
        out_specs=pl.BlockSpec((1, _BLK_ROWS, _H), lambda n, ib: (n, ib, 0)),
        out_shape=jax.ShapeDtypeStruct((_NH, _H, _H), f32),
    )(t_pad)
    return out[None]


# speedup vs baseline: 168.5459x; 1.3166x over previous
import jax
import jax.numpy as jnp
from jax.experimental import pallas as pl
from jax.experimental.pallas import tpu as pltpu

_H = 2048
_D = 512
_NH = 16
_TPAD = 2 * _H
_BLK_COLS = 1024
_BLK_ROWS = 512


def _mlp_kernel(c_ref, w0_ref, b0_ref, w1t_ref, o_ref):
    r = jnp.maximum(w0_ref[...] * c_ref[...] + b0_ref[...], 0.0)
    t = jax.lax.dot_general(
        w1t_ref[...], r, (((1,), (0,)), ((), ())),
        preferred_element_type=jnp.float32,
        precision=jax.lax.Precision.HIGHEST,
    )
    o_ref[...] = 16.0 * jax.nn.sigmoid(t)


_SLAB = _H + _BLK_ROWS


def _toeplitz_kernel(t_ref, o_ref):
    i0 = pl.program_id(1) * _BLK_ROWS
    slab = t_ref[0, :, pl.ds(_H - _BLK_ROWS - i0, _SLAB)]
    x = jnp.broadcast_to(slab, (_BLK_ROWS, _SLAB))
    y = pltpu.roll(x, _SLAB - (_BLK_ROWS - 1), axis=1, stride=1,
                   stride_axis=0)
    o_ref[...] = y[None, :, :_H]


def kernel(h, h2, bc, W0, b0, W1):
    f32 = jnp.float32
    coords_open = jnp.arange(-(_H - 1), _H, dtype=f32) / (h - 1)
    periodic_parts = jnp.concatenate([
        jnp.arange(1, _H // 2 + 1, dtype=f32),
        jnp.arange(-(_H // 2 - 1), _H // 2 + 1, dtype=f32),
        jnp.arange(-(_H // 2 - 1), 0, dtype=f32),
    ]) / (h - 1)
    pad_len = 2 * _H - 1 - periodic_parts.shape[0]
    coords_periodic = jnp.concatenate(
        [periodic_parts, jnp.zeros(pad_len, dtype=f32)])
    rel = jnp.where(bc == 1, coords_periodic, coords_open)
    c_pad = jnp.concatenate([rel, jnp.zeros(1, dtype=f32)]).reshape(1, _TPAD)

    w0c = W0.reshape(1, _D).T
    b0c = b0.reshape(_D, 1)
    w1t = W1.reshape(_D, _NH).T

    t_pad = pl.pallas_call(
        _mlp_kernel,
        grid=(_TPAD // _BLK_COLS,),
        in_specs=[
            pl.BlockSpec((1, _BLK_COLS), lambda j: (0, j)),
            pl.BlockSpec((_D, 1), lambda j: (0, 0)),
            pl.BlockSpec((_D, 1), lambda j: (0, 0)),
            pl.BlockSpec((_NH, _D), lambda j: (0, 0)),
        ],
        out_specs=pl.BlockSpec((_NH, _BLK_COLS), lambda j: (0, j)),
        out_shape=jax.ShapeDtypeStruct((_NH, _TPAD), f32),
    )(c_pad, w0c, b0c, w1t)
    t_pad = t_pad.reshape(_NH, 1, _TPAD)

    out = pl.pallas_call(
        _toeplitz_kernel,
        grid=(_NH, _H // _BLK_ROWS),
        in_specs=[pl.BlockSpec((1, 1, _TPAD), lambda n, ib: (n, 0, 0))],
        out_specs=pl.BlockSpec((1, _BLK_ROWS, _H), lambda n, ib: (n, ib, 0)),
        out_shape=jax.ShapeDtypeStruct((_NH, _H, _H), f32),
        compiler_params=pltpu.CompilerParams(
            dimension_semantics=("parallel", "parallel")),
    )(t_pad)
    return out[None]
